# Initial kernel scaffold; baseline (speedup 1.0000x reference)
import jax
import jax.numpy as jnp
from jax.experimental import pallas as pl


def kernel(x_main, x_mod, xyz, W_E1, W_E2, W_Q, W_K, W_V):
    raise NotImplementedError("write your pallas kernel here")


# speedup vs baseline: 1.0994x; 1.0994x over previous
import math

import jax
import jax.numpy as jnp
from jax.experimental import pallas as pl
from jax.experimental.pallas import tpu as pltpu

N = 8192
D_MAIN = 128
D_MOD = 128
D_H = 16
D_QK = 8
D_OUT = 128
TQ = 512
GRID = N // TQ


def _fused_kernel(x_main_ref, x_mod_ref, w_e1_ref, w_e2_ref, w_q_ref,
                  w_k_ref, w_v_ref, out_ref, h_s, k_s, v_s):
    i = pl.program_id(0)

    @pl.when(i == 0)
    def _prologue():
        w1a = w_e1_ref[0:D_MAIN, :]
        w1b = w_e1_ref[D_MAIN:D_MAIN + D_MOD, :]
        h1 = jnp.maximum(
            jnp.dot(x_main_ref[...], w1a, preferred_element_type=jnp.float32)
            + jnp.dot(x_mod_ref[...], w1b, preferred_element_type=jnp.float32),
            0.0)
        h = jnp.maximum(
            jnp.dot(h1, w_e2_ref[...], preferred_element_type=jnp.float32), 0.0)
        h_s[...] = h
        k_s[...] = jnp.dot(h, w_k_ref[...], preferred_element_type=jnp.float32)
        v_s[...] = jnp.dot(h, w_v_ref[...], preferred_element_type=jnp.float32)

    hq = h_s[pl.ds(i * TQ, TQ), :]
    q = jnp.dot(hq, w_q_ref[...], preferred_element_type=jnp.float32)
    scores = jax.lax.dot_general(
        q, k_s[...], (((1,), (1,)), ((), ())),
        preferred_element_type=jnp.float32) * (1.0 / math.sqrt(D_QK))
    m = jnp.max(scores, axis=1, keepdims=True)
    e = jnp.exp(scores - m)
    denom = jnp.sum(e, axis=1, keepdims=True)
    o = jnp.dot(e, v_s[...], preferred_element_type=jnp.float32)
    out_ref[...] = o / denom


def kernel(x_main, x_mod, xyz, W_E1, W_E2, W_Q, W_K, W_V):
    del xyz
    full = lambda s: pl.BlockSpec(s, lambda i: (0, 0))
    return pl.pallas_call(
        _fused_kernel,
        grid=(GRID,),
        in_specs=[
            full((N, D_MAIN)),
            full((N, D_MOD)),
            full((D_MAIN + D_MOD, D_H)),
            full((D_H, D_H)),
            full((D_H, D_QK)),
            full((D_H, D_QK)),
            full((D_H, D_OUT)),
        ],
        out_specs=pl.BlockSpec((TQ, D_OUT), lambda i: (i, 0)),
        out_shape=jax.ShapeDtypeStruct((N, D_OUT), jnp.float32),
        scratch_shapes=[
            pltpu.VMEM((N, D_H), jnp.float32),
            pltpu.VMEM((N, D_QK), jnp.float32),
            pltpu.VMEM((N, D_OUT), jnp.float32),
        ],
    )(x_main, x_mod, W_E1, W_E2, W_Q, W_K, W_V)
